# Initial kernel scaffold; baseline (speedup 1.0000x reference)
import functools

import jax
import jax.numpy as jnp
from jax import lax
from jax.experimental import pallas as pl
from jax.experimental.pallas import tpu as pltpu
from jax.experimental.pallas import tpu_sc as plsc

F = 32
NC = 2
NS = 16
NW = NC * NS
CH = 128


def _mm_bias_body(x_ref, w_ref, b_ref, o_ref):
    o_ref[...] = (
        jnp.dot(x_ref[...], w_ref[...], preferred_element_type=jnp.float32)
        + b_ref[...]
    )


def _mm_bias(x, w, b, block):
    m, k = x.shape
    f = w.shape[1]
    return pl.pallas_call(
        _mm_bias_body,
        grid=(m // block,),
        in_specs=[
            pl.BlockSpec((block, k), lambda i: (i, 0)),
            pl.BlockSpec((k, f), lambda i: (0, 0)),
            pl.BlockSpec((1, f), lambda i: (0, 0)),
        ],
        out_specs=pl.BlockSpec((block, f), lambda i: (i, 0)),
        out_shape=jax.ShapeDtypeStruct((m, f), jnp.float32),
    )(x, w, b)


def _edge_proj_body(e_ref, w1_ref, w2_ref, q1_ref, q2_ref):
    e = e_ref[...]
    q1_ref[...] = jnp.dot(e, w1_ref[...], preferred_element_type=jnp.float32)
    q2_ref[...] = jnp.dot(e, w2_ref[...], preferred_element_type=jnp.float32)


def _edge_proj(e_pad, w1, w2, block):
    m, k = e_pad.shape
    out = jax.ShapeDtypeStruct((m, F), jnp.float32)
    return pl.pallas_call(
        _edge_proj_body,
        grid=(m // block,),
        in_specs=[
            pl.BlockSpec((block, k), lambda i: (i, 0)),
            pl.BlockSpec((k, F), lambda i: (0, 0)),
            pl.BlockSpec((k, F), lambda i: (0, 0)),
        ],
        out_specs=[
            pl.BlockSpec((block, F), lambda i: (i, 0)),
            pl.BlockSpec((block, F), lambda i: (i, 0)),
        ],
        out_shape=[out, out],
    )(e_pad, w1, w2)


def _update_body(x_ref, a0_ref, a1_ref, wx_ref, wa_ref, bu_ref,
                 wm_ref, bm_ref, y_ref, p_ref):
    agg = a0_ref[...] + a1_ref[...]
    y = (
        jnp.dot(x_ref[...], wx_ref[...], preferred_element_type=jnp.float32)
        + jnp.dot(agg, wa_ref[...], preferred_element_type=jnp.float32)
        + bu_ref[...]
    )
    y_ref[...] = y
    p_ref[...] = (
        jnp.dot(y, wm_ref[...], preferred_element_type=jnp.float32)
        + bm_ref[...]
    )


def _update_and_project(x, a0, a1, wx, wa, bu, wm, bm, block):
    m, k = x.shape
    out = jax.ShapeDtypeStruct((m, F), jnp.float32)
    return pl.pallas_call(
        _update_body,
        grid=(m // block,),
        in_specs=[
            pl.BlockSpec((block, k), lambda i: (i, 0)),
            pl.BlockSpec((block, F), lambda i: (i, 0)),
            pl.BlockSpec((block, F), lambda i: (i, 0)),
            pl.BlockSpec((k, F), lambda i: (0, 0)),
            pl.BlockSpec((F, F), lambda i: (0, 0)),
            pl.BlockSpec((1, F), lambda i: (0, 0)),
            pl.BlockSpec((F, F), lambda i: (0, 0)),
            pl.BlockSpec((1, F), lambda i: (0, 0)),
        ],
        out_specs=[
            pl.BlockSpec((block, F), lambda i: (i, 0)),
            pl.BlockSpec((block, F), lambda i: (i, 0)),
        ],
        out_shape=[out, out],
    )(x, a0, a1, wx, wa, bu, wm, bm)


def _final_body(x_ref, a0_ref, a1_ref, bi_ref, wx_ref, wa_ref, bu_ref,
                wh_ref, bh_ref, wo_ref, bo_ref, o_ref, acc_ref):
    i = pl.program_id(0)
    nb = pl.num_programs(0)

    @pl.when(i == 0)
    def _():
        acc_ref[...] = jnp.zeros_like(acc_ref)

    agg = a0_ref[...] + a1_ref[...]
    y2 = (
        jnp.dot(x_ref[...], wx_ref[...], preferred_element_type=jnp.float32)
        + jnp.dot(agg, wa_ref[...], preferred_element_type=jnp.float32)
        + bu_ref[...]
    )
    bi = bi_ref[0, 0, :]
    onehot = (bi[:, None] == lax.broadcasted_iota(jnp.int32, (1, 64), 1))
    onehot = onehot.astype(jnp.float32)
    acc_ref[...] += lax.dot_general(
        onehot, y2, (((0,), (0,)), ((), ())),
        preferred_element_type=jnp.float32)

    @pl.when(i == nb - 1)
    def _():
        pooled = acc_ref[...]
        h = jnp.maximum(
            jnp.dot(pooled, wh_ref[...], preferred_element_type=jnp.float32)
            + bh_ref[...], 0.0)
        o_ref[...] = (
            jnp.dot(h, wo_ref[...], preferred_element_type=jnp.float32)
            + bo_ref[...]
        )


def _final(x, a0, a1, bidx3, wx, wa, bu, wh, bh, wo, bo, block):
    m, k = x.shape
    nb = m // block
    return pl.pallas_call(
        _final_body,
        grid=(nb,),
        in_specs=[
            pl.BlockSpec((block, k), lambda i: (i, 0)),
            pl.BlockSpec((block, F), lambda i: (i, 0)),
            pl.BlockSpec((block, F), lambda i: (i, 0)),
            pl.BlockSpec((1, 1, block), lambda i: (i, 0, 0)),
            pl.BlockSpec((k, F), lambda i: (0, 0)),
            pl.BlockSpec((F, F), lambda i: (0, 0)),
            pl.BlockSpec((1, F), lambda i: (0, 0)),
            pl.BlockSpec((F, F), lambda i: (0, 0)),
            pl.BlockSpec((1, F), lambda i: (0, 0)),
            pl.BlockSpec((F, 1), lambda i: (0, 0)),
            pl.BlockSpec((1, 1), lambda i: (0, 0)),
        ],
        out_specs=pl.BlockSpec((64, 1), lambda i: (0, 0)),
        out_shape=jax.ShapeDtypeStruct((64, 1), jnp.float32),
        scratch_shapes=[pltpu.VMEM((64, 32), jnp.float32)],
    )(x, a0, a1, bidx3, wx, wa, bu, wh, bh, wo, bo)


def _make_edge_pass(n_nodes, n_edges):
    n_chunks = n_edges // CH
    kmax = -(-n_chunks // NW)
    rows_per_sub = n_nodes // NS
    zb = 125
    n_zb = rows_per_sub // zb

    mesh = plsc.VectorSubcoreMesh(core_axis_name="c", subcore_axis_name="s")

    @functools.partial(
        pl.kernel,
        out_type=jax.ShapeDtypeStruct((2 * n_nodes, F), jnp.float32),
        mesh=mesh,
        scratch_types=[
            pltpu.VMEM((CH,), jnp.int32),
            pltpu.VMEM((1, CH), jnp.int32),
            pltpu.VMEM((CH, F), jnp.float32),
            pltpu.VMEM((125, F), jnp.float32),
            pltpu.VMEM_SHARED((n_nodes, F), jnp.float32),
            pltpu.SemaphoreType.DMA,
        ],
    )
    def edge_pass(p_hbm, q_hbm, src_hbm, dst_hbm, out_hbm,
                  src_v, dst_v, m_v, z_v, acc_sh, sem):
        c = lax.axis_index("c")
        s = lax.axis_index("s")
        wid = s * NC + c
        zb_rows = 125

        def zfill(j, _):
            z_v[j, pl.ds(0, 16)] = jnp.zeros((16,), jnp.float32)
            z_v[j, pl.ds(16, 16)] = jnp.zeros((16,), jnp.float32)
            return 0
        lax.fori_loop(0, zb_rows, zfill, 0)

        def zcopy(i, _):
            row0 = s * rows_per_sub + i * zb_rows
            pltpu.sync_copy(z_v, acc_sh.at[pl.ds(row0, zb_rows)])
            return 0
        lax.fori_loop(0, n_zb, zcopy, 0)
        plsc.subcore_barrier()

        def chunk_body(k, _):
            chunk = wid + k * NW

            @pl.when(chunk < n_chunks)
            def _():
                base = chunk * CH
                pltpu.sync_copy(src_hbm.at[pl.ds(base, CH)], src_v)
                pltpu.sync_copy(dst_hbm.at[pl.ds(base, CH)], dst_v.at[0])
                pltpu.sync_copy(q_hbm.at[pl.ds(base, CH)], m_v)
                pltpu.async_copy(p_hbm.at[src_v], m_v, sem, add=True).wait()

                def relu_body(j, _):
                    for h in range(2):
                        sl = pl.ds(h * 16, 16)
                        m_v[j, sl] = jnp.maximum(m_v[j, sl], 0.0)
                    return 0
                lax.fori_loop(0, CH, relu_body, 0)

                pltpu.sync_copy(m_v, acc_sh.at[dst_v.at[0]], add=True)
            return 0
        lax.fori_loop(0, kmax, chunk_body, 0)
        plsc.subcore_barrier()

        def fcopy(i, _):
            row0 = s * rows_per_sub + i * zb_rows
            pltpu.sync_copy(acc_sh.at[pl.ds(row0, zb_rows)], z_v)
            pltpu.sync_copy(z_v, out_hbm.at[pl.ds(c * n_nodes + row0, zb_rows)])
            return 0
        lax.fori_loop(0, n_zb, fcopy, 0)

    return edge_pass


def kernel(node_features, edge_features, edge_idx, batch_idx,
           WM1, bM1, WU1, bU1, WM2, bM2, WU2, bU2, Wh, bh, Wo, bo):
    n = node_features.shape[0]
    e = edge_features.shape[0]
    src = edge_idx[0]
    dst = edge_idx[1]

    x_pad = jnp.pad(node_features, ((0, 0), (0, 3)))
    e_pad = jnp.pad(edge_features, ((0, 0), (0, 5)))
    wq1 = jnp.pad(WM1[21:24], ((0, 5), (0, 0)))
    wq2 = jnp.pad(WM2[32:35], ((0, 5), (0, 0)))
    wu1x = jnp.pad(WU1[:21], ((0, 3), (0, 0)))

    nblk = 2000
    eblk = 2000

    edge_pass = _make_edge_pass(n, e)

    p1 = _mm_bias(x_pad, WM1, bM1[None, :], nblk)
    q1, q2 = _edge_proj(e_pad, wq1, wq2, eblk)
    agg1 = edge_pass(p1, q1, src, dst)
    y, p2 = _update_and_project(
        x_pad, agg1[:n], agg1[n:], wu1x, WU1[21:53], bU1[None, :],
        WM2[:32], bM2[None, :], nblk)

    agg2 = edge_pass(p2, q2, src, dst)

    bidx3 = batch_idx.reshape(n // nblk, 1, nblk)
    out = _final(
        y, agg2[:n], agg2[n:], bidx3, WU2[:32], WU2[32:64], bU2[None, :],
        Wh, bh[None, :], Wo, bo[None, None, 0:1], nblk)
    return out


# speedup vs baseline: 1.5381x; 1.5381x over previous
import functools

import jax
import jax.numpy as jnp
from jax import lax
from jax.experimental import pallas as pl
from jax.experimental.pallas import tpu as pltpu
from jax.experimental.pallas import tpu_sc as plsc

F = 32
NC = 2
NS = 16
NW = NC * NS
CH = 128


def _mm_bias_body(x_ref, w_ref, b_ref, o_ref):
    o_ref[...] = (
        jnp.dot(x_ref[...], w_ref[...], preferred_element_type=jnp.float32)
        + b_ref[...]
    )


def _mm_bias(x, w, b, block):
    m, k = x.shape
    f = w.shape[1]
    return pl.pallas_call(
        _mm_bias_body,
        grid=(m // block,),
        in_specs=[
            pl.BlockSpec((block, k), lambda i: (i, 0)),
            pl.BlockSpec((k, f), lambda i: (0, 0)),
            pl.BlockSpec((1, f), lambda i: (0, 0)),
        ],
        out_specs=pl.BlockSpec((block, f), lambda i: (i, 0)),
        out_shape=jax.ShapeDtypeStruct((m, f), jnp.float32),
    )(x, w, b)


def _edge_proj_body(e_ref, w1_ref, w2_ref, q1_ref, q2_ref):
    e = e_ref[...]
    q1_ref[...] = jnp.dot(e, w1_ref[...], preferred_element_type=jnp.float32)
    q2_ref[...] = jnp.dot(e, w2_ref[...], preferred_element_type=jnp.float32)


def _edge_proj(e_pad, w1, w2, block):
    m, k = e_pad.shape
    out = jax.ShapeDtypeStruct((m, F), jnp.float32)
    return pl.pallas_call(
        _edge_proj_body,
        grid=(m // block,),
        in_specs=[
            pl.BlockSpec((block, k), lambda i: (i, 0)),
            pl.BlockSpec((k, F), lambda i: (0, 0)),
            pl.BlockSpec((k, F), lambda i: (0, 0)),
        ],
        out_specs=[
            pl.BlockSpec((block, F), lambda i: (i, 0)),
            pl.BlockSpec((block, F), lambda i: (i, 0)),
        ],
        out_shape=[out, out],
    )(e_pad, w1, w2)


def _update_body(x_ref, a0_ref, a1_ref, wx_ref, wa_ref, bu_ref,
                 wm_ref, bm_ref, y_ref, p_ref):
    agg = a0_ref[...] + a1_ref[...]
    y = (
        jnp.dot(x_ref[...], wx_ref[...], preferred_element_type=jnp.float32)
        + jnp.dot(agg, wa_ref[...], preferred_element_type=jnp.float32)
        + bu_ref[...]
    )
    y_ref[...] = y
    p_ref[...] = (
        jnp.dot(y, wm_ref[...], preferred_element_type=jnp.float32)
        + bm_ref[...]
    )


def _update_and_project(x, a0, a1, wx, wa, bu, wm, bm, block):
    m, k = x.shape
    out = jax.ShapeDtypeStruct((m, F), jnp.float32)
    return pl.pallas_call(
        _update_body,
        grid=(m // block,),
        in_specs=[
            pl.BlockSpec((block, k), lambda i: (i, 0)),
            pl.BlockSpec((block, F), lambda i: (i, 0)),
            pl.BlockSpec((block, F), lambda i: (i, 0)),
            pl.BlockSpec((k, F), lambda i: (0, 0)),
            pl.BlockSpec((F, F), lambda i: (0, 0)),
            pl.BlockSpec((1, F), lambda i: (0, 0)),
            pl.BlockSpec((F, F), lambda i: (0, 0)),
            pl.BlockSpec((1, F), lambda i: (0, 0)),
        ],
        out_specs=[
            pl.BlockSpec((block, F), lambda i: (i, 0)),
            pl.BlockSpec((block, F), lambda i: (i, 0)),
        ],
        out_shape=[out, out],
    )(x, a0, a1, wx, wa, bu, wm, bm)


def _final_body(x_ref, a0_ref, a1_ref, bi_ref, wx_ref, wa_ref, bu_ref,
                wh_ref, bh_ref, wo_ref, bo_ref, o_ref, acc_ref):
    i = pl.program_id(0)
    nb = pl.num_programs(0)

    @pl.when(i == 0)
    def _():
        acc_ref[...] = jnp.zeros_like(acc_ref)

    agg = a0_ref[...] + a1_ref[...]
    y2 = (
        jnp.dot(x_ref[...], wx_ref[...], preferred_element_type=jnp.float32)
        + jnp.dot(agg, wa_ref[...], preferred_element_type=jnp.float32)
        + bu_ref[...]
    )
    bi = bi_ref[0, 0, :]
    onehot = (bi[:, None] == lax.broadcasted_iota(jnp.int32, (1, 64), 1))
    onehot = onehot.astype(jnp.float32)
    acc_ref[...] += lax.dot_general(
        onehot, y2, (((0,), (0,)), ((), ())),
        preferred_element_type=jnp.float32)

    @pl.when(i == nb - 1)
    def _():
        pooled = acc_ref[...]
        h = jnp.maximum(
            jnp.dot(pooled, wh_ref[...], preferred_element_type=jnp.float32)
            + bh_ref[...], 0.0)
        o_ref[...] = (
            jnp.dot(h, wo_ref[...], preferred_element_type=jnp.float32)
            + bo_ref[...]
        )


def _final(x, a0, a1, bidx3, wx, wa, bu, wh, bh, wo, bo, block):
    m, k = x.shape
    nb = m // block
    return pl.pallas_call(
        _final_body,
        grid=(nb,),
        in_specs=[
            pl.BlockSpec((block, k), lambda i: (i, 0)),
            pl.BlockSpec((block, F), lambda i: (i, 0)),
            pl.BlockSpec((block, F), lambda i: (i, 0)),
            pl.BlockSpec((1, 1, block), lambda i: (i, 0, 0)),
            pl.BlockSpec((k, F), lambda i: (0, 0)),
            pl.BlockSpec((F, F), lambda i: (0, 0)),
            pl.BlockSpec((1, F), lambda i: (0, 0)),
            pl.BlockSpec((F, F), lambda i: (0, 0)),
            pl.BlockSpec((1, F), lambda i: (0, 0)),
            pl.BlockSpec((F, 1), lambda i: (0, 0)),
            pl.BlockSpec((1, 1), lambda i: (0, 0)),
        ],
        out_specs=pl.BlockSpec((64, 1), lambda i: (0, 0)),
        out_shape=jax.ShapeDtypeStruct((64, 1), jnp.float32),
        scratch_shapes=[pltpu.VMEM((64, 32), jnp.float32)],
    )(x, a0, a1, bidx3, wx, wa, bu, wh, bh, wo, bo)


def _make_edge_pass(n_nodes, n_edges):
    n_chunks = n_edges // CH
    kmax = -(-n_chunks // NW)
    zb = 200
    n_zb = n_nodes // zb
    zmax = -(-n_zb // NS)

    mesh = plsc.VectorSubcoreMesh(core_axis_name="c", subcore_axis_name="s")

    @functools.partial(
        pl.kernel,
        out_type=jax.ShapeDtypeStruct((2 * n_nodes, F), jnp.float32),
        mesh=mesh,
        scratch_types=[
            pltpu.VMEM((CH,), jnp.int32),
            pltpu.VMEM((1, CH), jnp.int32),
            pltpu.VMEM((CH, F), jnp.float32),
            pltpu.VMEM((200, F), jnp.float32),
            pltpu.VMEM_SHARED((n_nodes, F), jnp.float32),
            pltpu.SemaphoreType.DMA,
        ],
        compiler_params=pltpu.CompilerParams(use_tc_tiling_on_sc=False),
    )
    def edge_pass(p_hbm, q_hbm, src_hbm, dst_hbm, out_hbm,
                  src_v, dst_v, m_v, z_v, acc_sh, sem):
        c = lax.axis_index("c")
        s = lax.axis_index("s")
        wid = s * NC + c

        def zfill(j, _):
            z_v[j, pl.ds(0, 16)] = jnp.zeros((16,), jnp.float32)
            z_v[j, pl.ds(16, 16)] = jnp.zeros((16,), jnp.float32)
            return 0
        lax.fori_loop(0, zb, zfill, 0)

        def zcopy(i, _):
            blk = s + i * NS

            @pl.when(blk < n_zb)
            def _():
                pltpu.sync_copy(z_v, acc_sh.at[pl.ds(blk * zb, zb)])
            return 0
        lax.fori_loop(0, zmax, zcopy, 0)
        plsc.subcore_barrier()

        def chunk_body(k, _):
            chunk = wid + k * NW

            @pl.when(chunk < n_chunks)
            def _():
                base = chunk * CH
                pltpu.sync_copy(src_hbm.at[pl.ds(base, CH)], src_v)
                pltpu.sync_copy(dst_hbm.at[pl.ds(base, CH)], dst_v.at[0])
                pltpu.sync_copy(q_hbm.at[pl.ds(base, CH)], m_v)
                pltpu.async_copy(p_hbm.at[src_v], m_v, sem, add=True).wait()

                def relu_body(j, _):
                    for h in range(2):
                        sl = pl.ds(h * 16, 16)
                        m_v[j, sl] = jnp.maximum(m_v[j, sl], 0.0)
                    return 0
                lax.fori_loop(0, CH, relu_body, 0)

                pltpu.sync_copy(m_v, acc_sh.at[dst_v.at[0]], add=True)
            return 0
        lax.fori_loop(0, kmax, chunk_body, 0)
        plsc.subcore_barrier()

        def fcopy(i, _):
            blk = s + i * NS

            @pl.when(blk < n_zb)
            def _():
                row0 = blk * zb
                pltpu.sync_copy(acc_sh.at[pl.ds(row0, zb)], z_v)
                pltpu.sync_copy(z_v, out_hbm.at[pl.ds(c * n_nodes + row0, zb)])
            return 0
        lax.fori_loop(0, zmax, fcopy, 0)

    return edge_pass


def kernel(node_features, edge_features, edge_idx, batch_idx,
           WM1, bM1, WU1, bU1, WM2, bM2, WU2, bU2, Wh, bh, Wo, bo):
    n = node_features.shape[0]
    e = edge_features.shape[0]
    src = edge_idx[0]
    dst = edge_idx[1]

    x_pad = jnp.pad(node_features, ((0, 0), (0, 3)))
    e_pad = jnp.pad(edge_features, ((0, 0), (0, 5)))
    wq1 = jnp.pad(WM1[21:24], ((0, 5), (0, 0)))
    wq2 = jnp.pad(WM2[32:35], ((0, 5), (0, 0)))
    wu1x = jnp.pad(WU1[:21], ((0, 3), (0, 0)))

    nblk = 2000
    eblk = 2000

    edge_pass = _make_edge_pass(n, e)

    p1 = _mm_bias(x_pad, WM1, bM1[None, :], nblk)
    q1, q2 = _edge_proj(e_pad, wq1, wq2, eblk)
    agg1 = edge_pass(p1, q1, src, dst)
    y, p2 = _update_and_project(
        x_pad, agg1[:n], agg1[n:], wu1x, WU1[21:53], bU1[None, :],
        WM2[:32], bM2[None, :], nblk)

    agg2 = edge_pass(p2, q2, src, dst)

    bidx3 = batch_idx.reshape(n // nblk, 1, nblk)
    out = _final(
        y, agg2[:n], agg2[n:], bidx3, WU2[:32], WU2[32:64], bU2[None, :],
        Wh, bh[None, :], Wo, bo[None, :], nblk)
    return out


# speedup vs baseline: 1.8302x; 1.1899x over previous
import functools

import jax
import jax.numpy as jnp
from jax import lax
from jax.experimental import pallas as pl
from jax.experimental.pallas import tpu as pltpu
from jax.experimental.pallas import tpu_sc as plsc

F = 32
NC = 2
NS = 16
NW = NC * NS
CH = 128


def _mm_bias_body(x_ref, w_ref, b_ref, o_ref):
    o_ref[...] = (
        jnp.dot(x_ref[...], w_ref[...], preferred_element_type=jnp.float32)
        + b_ref[...]
    )


def _mm_bias(x, w, b, block):
    m, k = x.shape
    f = w.shape[1]
    return pl.pallas_call(
        _mm_bias_body,
        grid=(m // block,),
        in_specs=[
            pl.BlockSpec((block, k), lambda i: (i, 0)),
            pl.BlockSpec((k, f), lambda i: (0, 0)),
            pl.BlockSpec((1, f), lambda i: (0, 0)),
        ],
        out_specs=pl.BlockSpec((block, f), lambda i: (i, 0)),
        out_shape=jax.ShapeDtypeStruct((m, f), jnp.float32),
    )(x, w, b)


def _edge_proj_body(e_ref, w1_ref, w2_ref, q1_ref, q2_ref, *, n_valid, block):
    i = pl.program_id(0)
    e = e_ref[...]
    rows = i * block + lax.broadcasted_iota(jnp.int32, (block, 1), 0)
    mask = rows < n_valid
    neg = jnp.float32(-1e30)
    q1 = jnp.dot(e, w1_ref[...], preferred_element_type=jnp.float32)
    q2 = jnp.dot(e, w2_ref[...], preferred_element_type=jnp.float32)
    q1_ref[...] = jnp.where(mask, q1, neg)
    q2_ref[...] = jnp.where(mask, q2, neg)


def _edge_proj(e_pad, w1, w2, n_valid, block):
    m, k = e_pad.shape
    out = jax.ShapeDtypeStruct((m, F), jnp.float32)
    return pl.pallas_call(
        functools.partial(_edge_proj_body, n_valid=n_valid, block=block),
        grid=(m // block,),
        in_specs=[
            pl.BlockSpec((block, k), lambda i: (i, 0)),
            pl.BlockSpec((k, F), lambda i: (0, 0)),
            pl.BlockSpec((k, F), lambda i: (0, 0)),
        ],
        out_specs=[
            pl.BlockSpec((block, F), lambda i: (i, 0)),
            pl.BlockSpec((block, F), lambda i: (i, 0)),
        ],
        out_shape=[out, out],
    )(e_pad, w1, w2)


def _update_body(x_ref, a0_ref, a1_ref, wx_ref, wa_ref, bu_ref,
                 wm_ref, bm_ref, y_ref, p_ref):
    agg = a0_ref[...] + a1_ref[...]
    y = (
        jnp.dot(x_ref[...], wx_ref[...], preferred_element_type=jnp.float32)
        + jnp.dot(agg, wa_ref[...], preferred_element_type=jnp.float32)
        + bu_ref[...]
    )
    y_ref[...] = y
    p_ref[...] = (
        jnp.dot(y, wm_ref[...], preferred_element_type=jnp.float32)
        + bm_ref[...]
    )


def _update_and_project(x, a0, a1, wx, wa, bu, wm, bm, block):
    m, k = x.shape
    out = jax.ShapeDtypeStruct((m, F), jnp.float32)
    return pl.pallas_call(
        _update_body,
        grid=(m // block,),
        in_specs=[
            pl.BlockSpec((block, k), lambda i: (i, 0)),
            pl.BlockSpec((block, F), lambda i: (i, 0)),
            pl.BlockSpec((block, F), lambda i: (i, 0)),
            pl.BlockSpec((k, F), lambda i: (0, 0)),
            pl.BlockSpec((F, F), lambda i: (0, 0)),
            pl.BlockSpec((1, F), lambda i: (0, 0)),
            pl.BlockSpec((F, F), lambda i: (0, 0)),
            pl.BlockSpec((1, F), lambda i: (0, 0)),
        ],
        out_specs=[
            pl.BlockSpec((block, F), lambda i: (i, 0)),
            pl.BlockSpec((block, F), lambda i: (i, 0)),
        ],
        out_shape=[out, out],
    )(x, a0, a1, wx, wa, bu, wm, bm)


def _final_body(x_ref, a0_ref, a1_ref, bi_ref, wx_ref, wa_ref, bu_ref,
                wh_ref, bh_ref, wo_ref, bo_ref, o_ref, acc_ref):
    i = pl.program_id(0)
    nb = pl.num_programs(0)

    @pl.when(i == 0)
    def _():
        acc_ref[...] = jnp.zeros_like(acc_ref)

    agg = a0_ref[...] + a1_ref[...]
    y2 = (
        jnp.dot(x_ref[...], wx_ref[...], preferred_element_type=jnp.float32)
        + jnp.dot(agg, wa_ref[...], preferred_element_type=jnp.float32)
        + bu_ref[...]
    )
    bi = bi_ref[0, 0, :]
    onehot = (bi[:, None] == lax.broadcasted_iota(jnp.int32, (1, 64), 1))
    onehot = onehot.astype(jnp.float32)
    acc_ref[...] += lax.dot_general(
        onehot, y2, (((0,), (0,)), ((), ())),
        preferred_element_type=jnp.float32)

    @pl.when(i == nb - 1)
    def _():
        pooled = acc_ref[...]
        h = jnp.maximum(
            jnp.dot(pooled, wh_ref[...], preferred_element_type=jnp.float32)
            + bh_ref[...], 0.0)
        o_ref[...] = (
            jnp.dot(h, wo_ref[...], preferred_element_type=jnp.float32)
            + bo_ref[...]
        )


def _final(x, a0, a1, bidx3, wx, wa, bu, wh, bh, wo, bo, block):
    m, k = x.shape
    nb = m // block
    return pl.pallas_call(
        _final_body,
        grid=(nb,),
        in_specs=[
            pl.BlockSpec((block, k), lambda i: (i, 0)),
            pl.BlockSpec((block, F), lambda i: (i, 0)),
            pl.BlockSpec((block, F), lambda i: (i, 0)),
            pl.BlockSpec((1, 1, block), lambda i: (i, 0, 0)),
            pl.BlockSpec((k, F), lambda i: (0, 0)),
            pl.BlockSpec((F, F), lambda i: (0, 0)),
            pl.BlockSpec((1, F), lambda i: (0, 0)),
            pl.BlockSpec((F, F), lambda i: (0, 0)),
            pl.BlockSpec((1, F), lambda i: (0, 0)),
            pl.BlockSpec((F, 1), lambda i: (0, 0)),
            pl.BlockSpec((1, 1), lambda i: (0, 0)),
        ],
        out_specs=pl.BlockSpec((64, 1), lambda i: (0, 0)),
        out_shape=jax.ShapeDtypeStruct((64, 1), jnp.float32),
        scratch_shapes=[pltpu.VMEM((64, 32), jnp.float32)],
    )(x, a0, a1, bidx3, wx, wa, bu, wh, bh, wo, bo)


NSUB = 2
SUP = NSUB * CH


def _make_edge_pass(n_nodes, n_edges_pad):
    n_sup = n_edges_pad // SUP
    m_sup = -(-n_sup // NW)
    kloop = (m_sup + 2) // 2
    zb = 80
    n_zb = n_nodes // zb
    zmax = -(-n_zb // NS)

    mesh = plsc.VectorSubcoreMesh(core_axis_name="c", subcore_axis_name="s")

    @functools.partial(
        pl.kernel,
        out_type=jax.ShapeDtypeStruct((2 * n_nodes, F), jnp.float32),
        mesh=mesh,
        scratch_types=[
            pltpu.VMEM((SUP,), jnp.int32),
            pltpu.VMEM((SUP,), jnp.int32),
            pltpu.VMEM((NSUB, CH), jnp.int32),
            pltpu.VMEM((NSUB, CH), jnp.int32),
            pltpu.VMEM((SUP, F), jnp.float32),
            pltpu.VMEM((SUP, F), jnp.float32),
            pltpu.VMEM((80, F), jnp.float32),
            pltpu.VMEM_SHARED((n_nodes, F), jnp.float32),
            pltpu.SemaphoreType.DMA,
            pltpu.SemaphoreType.DMA,
            pltpu.SemaphoreType.DMA,
            pltpu.SemaphoreType.DMA,
        ],
        compiler_params=pltpu.CompilerParams(use_tc_tiling_on_sc=False),
    )
    def edge_pass(p_hbm, q_hbm, src_hbm, dst_hbm, out_hbm,
                  sv0, sv1, dv0, dv1, mv0, mv1, z_v, acc_sh,
                  ls0, ls1, gs0, gs1):
        c = lax.axis_index("c")
        s = lax.axis_index("s")
        wid = s * NC + c
        sv = (sv0, sv1)
        dv = (dv0, dv1)
        mv = (mv0, mv1)
        ls = (ls0, ls1)
        gs = (gs0, gs1)

        def zfill(j, _):
            z_v[j, pl.ds(0, 16)] = jnp.zeros((16,), jnp.float32)
            z_v[j, pl.ds(16, 16)] = jnp.zeros((16,), jnp.float32)
            return 0
        lax.fori_loop(0, zb, zfill, 0)

        def zcopy(i, _):
            blk = s + i * NS

            @pl.when(blk < n_zb)
            def _():
                pltpu.sync_copy(z_v, acc_sh.at[pl.ds(blk * zb, zb)])
            return 0
        lax.fori_loop(0, zmax, zcopy, 0)
        plsc.subcore_barrier()

        def lin_cps(sup, b):
            base = sup * SUP
            cps = [
                pltpu.make_async_copy(
                    src_hbm.at[pl.ds(base, SUP)], sv[b], ls[b]),
                pltpu.make_async_copy(
                    q_hbm.at[pl.ds(base, SUP)], mv[b], ls[b]),
            ]
            for j in range(NSUB):
                cps.append(pltpu.make_async_copy(
                    dst_hbm.at[pl.ds(base + j * CH, CH)], dv[b].at[j], ls[b]))
            return cps

        def g_cps(b):
            return [
                pltpu.make_async_copy(
                    p_hbm.at[sv[b].at[pl.ds(j * CH, CH)]],
                    mv[b].at[pl.ds(j * CH, CH)], gs[b])
                for j in range(NSUB)
            ]

        def stage(i, a):
            b = 1 - a
            sup_a = wid + i * NW

            @pl.when(sup_a < n_sup)
            def _():
                for cp in lin_cps(sup_a, a):
                    cp.wait()
                for cp in g_cps(a):
                    cp.start(add=True)

            sup_b = wid + (i - 1) * NW

            @pl.when(jnp.logical_and(i >= 1, sup_b < n_sup))
            def _():
                for cp in g_cps(b):
                    cp.wait()

                def relu_body(j, _):
                    for h in range(2):
                        sl = pl.ds(h * 16, 16)
                        mv[b][j, sl] = jnp.maximum(mv[b][j, sl], 0.0)
                    return 0
                lax.fori_loop(0, SUP, relu_body, 0, unroll=8)
                for j in range(NSUB):
                    pltpu.sync_copy(
                        mv[b].at[pl.ds(j * CH, CH)],
                        acc_sh.at[dv[b].at[j]], add=True)

            sup_c = wid + (i + 1) * NW

            @pl.when(sup_c < n_sup)
            def _():
                for cp in lin_cps(sup_c, b):
                    cp.start()

        @pl.when(wid < n_sup)
        def _():
            for cp in lin_cps(wid, 0):
                cp.start()

        def loop_body(k, _):
            stage(2 * k, 0)
            stage(2 * k + 1, 1)
            return 0
        lax.fori_loop(0, kloop, loop_body, 0)
        plsc.subcore_barrier()

        def fcopy(i, _):
            blk = s + i * NS

            @pl.when(blk < n_zb)
            def _():
                row0 = blk * zb
                pltpu.sync_copy(acc_sh.at[pl.ds(row0, zb)], z_v)
                pltpu.sync_copy(z_v, out_hbm.at[pl.ds(c * n_nodes + row0, zb)])
            return 0
        lax.fori_loop(0, zmax, fcopy, 0)

    return edge_pass


def kernel(node_features, edge_features, edge_idx, batch_idx,
           WM1, bM1, WU1, bU1, WM2, bM2, WU2, bU2, Wh, bh, Wo, bo):
    n = node_features.shape[0]
    e = edge_features.shape[0]
    e_pad_len = -(-e // SUP) * SUP
    idx_pad = jnp.pad(edge_idx, ((0, 0), (0, e_pad_len - e)))
    src = idx_pad[0]
    dst = idx_pad[1]

    x_pad = jnp.pad(node_features, ((0, 0), (0, 3)))
    e_feat = jnp.pad(edge_features, ((0, 0), (0, 5)),
                     constant_values=0.0)
    e_feat = jnp.pad(e_feat, ((0, e_pad_len - e), (0, 0)))
    wq1 = jnp.pad(WM1[21:24], ((0, 5), (0, 0)))
    wq2 = jnp.pad(WM2[32:35], ((0, 5), (0, 0)))
    wu1x = jnp.pad(WU1[:21], ((0, 3), (0, 0)))

    nblk = 2000
    eblk = 1536

    edge_pass = _make_edge_pass(n, e_pad_len)

    p1 = _mm_bias(x_pad, WM1, bM1[None, :], nblk)
    q1, q2 = _edge_proj(e_feat, wq1, wq2, e, eblk)
    agg1 = edge_pass(p1, q1, src, dst)
    y, p2 = _update_and_project(
        x_pad, agg1[:n], agg1[n:], wu1x, WU1[21:53], bU1[None, :],
        WM2[:32], bM2[None, :], nblk)

    agg2 = edge_pass(p2, q2, src, dst)

    bidx3 = batch_idx.reshape(n // nblk, 1, nblk)
    out = _final(
        y, agg2[:n], agg2[n:], bidx3, WU2[:32], WU2[32:64], bU2[None, :],
        Wh, bh[None, :], Wo, bo[None, :], nblk)
    return out


# speedup vs baseline: 9.0862x; 4.9646x over previous
import functools

import jax
import jax.numpy as jnp
from jax import lax
from jax.experimental import pallas as pl
from jax.experimental.pallas import tpu as pltpu
from jax.experimental.pallas import tpu_sc as plsc

F = 32
NC = 2
NS = 16
NW = NC * NS
CH = 128
NSUB = 2
SUP = NSUB * CH


def _mm_bias_body(x_ref, w_ref, b_ref, o_ref):
    o_ref[...] = (
        jnp.dot(x_ref[...], w_ref[...], preferred_element_type=jnp.float32)
        + b_ref[...]
    )


def _mm_bias(x, w, b):
    m, k = x.shape
    f = w.shape[1]
    return pl.pallas_call(
        _mm_bias_body,
        grid=(1,),
        in_specs=[
            pl.BlockSpec((m, k), lambda i: (0, 0)),
            pl.BlockSpec((k, f), lambda i: (0, 0)),
            pl.BlockSpec((1, f), lambda i: (0, 0)),
        ],
        out_specs=pl.BlockSpec((m, f), lambda i: (0, 0)),
        out_shape=jax.ShapeDtypeStruct((m, f), jnp.float32),
    )(x, w, b)


def _update_body(x_ref, a0_ref, a1_ref, wx_ref, wa_ref, bu_ref,
                 wm_ref, bm_ref, y_ref, p_ref):
    agg = a0_ref[0] + a1_ref[0]
    y = (
        jnp.dot(x_ref[...], wx_ref[...], preferred_element_type=jnp.float32)
        + jnp.dot(agg, wa_ref[...], preferred_element_type=jnp.float32)
        + bu_ref[...]
    )
    y_ref[...] = y
    p_ref[...] = (
        jnp.dot(y, wm_ref[...], preferred_element_type=jnp.float32)
        + bm_ref[...]
    )


def _update_and_project(x, aggp, wx, wa, bu, wm, bm):
    m, k = x.shape
    out = jax.ShapeDtypeStruct((m, k), jnp.float32)
    return pl.pallas_call(
        _update_body,
        grid=(1,),
        in_specs=[
            pl.BlockSpec((m, k), lambda i: (0, 0)),
            pl.BlockSpec((1, m, k), lambda i: (0, 0, 0)),
            pl.BlockSpec((1, m, k), lambda i: (1, 0, 0)),
            pl.BlockSpec((k, k), lambda i: (0, 0)),
            pl.BlockSpec((k, k), lambda i: (0, 0)),
            pl.BlockSpec((1, k), lambda i: (0, 0)),
            pl.BlockSpec((k, k), lambda i: (0, 0)),
            pl.BlockSpec((1, k), lambda i: (0, 0)),
        ],
        out_specs=[
            pl.BlockSpec((m, k), lambda i: (0, 0)),
            pl.BlockSpec((m, k), lambda i: (0, 0)),
        ],
        out_shape=[out, out],
    )(x, aggp, aggp, wx, wa, bu, wm, bm)


def _final_body(y_ref, a0_ref, a1_ref, oh0_ref, oh1_ref, oh2_ref, oh3_ref,
                wx_ref, wa_ref, bu_ref, wh_ref, bh_ref, wo_ref, bo_ref,
                o_ref):
    agg = a0_ref[0] + a1_ref[0]
    y2 = (
        jnp.dot(y_ref[...], wx_ref[...], preferred_element_type=jnp.float32)
        + jnp.dot(agg, wa_ref[...], preferred_element_type=jnp.float32)
        + bu_ref[...]
    )
    ohs = (oh0_ref, oh1_ref, oh2_ref, oh3_ref)
    pooled = jnp.zeros((64, F), jnp.float32)
    for k in range(4):
        pooled += lax.dot_general(
            ohs[k][...], y2[:, k * F:(k + 1) * F],
            (((0,), (0,)), ((), ())), preferred_element_type=jnp.float32)
    h = jnp.maximum(
        jnp.dot(pooled, wh_ref[...], preferred_element_type=jnp.float32)
        + bh_ref[...], 0.0)
    o_ref[...] = (
        jnp.dot(h, wo_ref[...], preferred_element_type=jnp.float32)
        + bo_ref[...]
    )


def _final(y, aggp, oh, wx, wa, bu, wh, bh, wo, bo):
    m, k = y.shape
    return pl.pallas_call(
        _final_body,
        grid=(1,),
        in_specs=[
            pl.BlockSpec((m, k), lambda i: (0, 0)),
            pl.BlockSpec((1, m, k), lambda i: (0, 0, 0)),
            pl.BlockSpec((1, m, k), lambda i: (1, 0, 0)),
            pl.BlockSpec((m, 64), lambda i: (0, 0)),
            pl.BlockSpec((m, 64), lambda i: (0, 0)),
            pl.BlockSpec((m, 64), lambda i: (0, 0)),
            pl.BlockSpec((m, 64), lambda i: (0, 0)),
            pl.BlockSpec((k, k), lambda i: (0, 0)),
            pl.BlockSpec((k, k), lambda i: (0, 0)),
            pl.BlockSpec((1, k), lambda i: (0, 0)),
            pl.BlockSpec((F, F), lambda i: (0, 0)),
            pl.BlockSpec((1, F), lambda i: (0, 0)),
            pl.BlockSpec((F, 1), lambda i: (0, 0)),
            pl.BlockSpec((1, 1), lambda i: (0, 0)),
        ],
        out_specs=pl.BlockSpec((64, 1), lambda i: (0, 0)),
        out_shape=jax.ShapeDtypeStruct((64, 1), jnp.float32),
    )(y, aggp, aggp, oh[0], oh[1], oh[2], oh[3],
      wx, wa, bu, wh, bh, wo, bo)


def _make_edge_pass(n_nodes, n_pad, n_edges):
    n_sup = n_edges // SUP
    m_sup = -(-n_sup // NW)
    kloop = (m_sup + 2) // 2
    zb = 80
    n_zb = n_nodes // zb
    zmax = -(-n_zb // NS)

    mesh = plsc.VectorSubcoreMesh(core_axis_name="c", subcore_axis_name="s")

    @functools.partial(
        pl.kernel,
        out_type=jax.ShapeDtypeStruct((2 * n_pad, F), jnp.float32),
        mesh=mesh,
        scratch_types=[
            pltpu.VMEM((SUP,), jnp.int32),
            pltpu.VMEM((SUP,), jnp.int32),
            pltpu.VMEM((NSUB, CH), jnp.int32),
            pltpu.VMEM((NSUB, CH), jnp.int32),
            pltpu.VMEM((SUP,), jnp.float32),
            pltpu.VMEM((SUP,), jnp.float32),
            pltpu.VMEM((SUP,), jnp.float32),
            pltpu.VMEM((SUP,), jnp.float32),
            pltpu.VMEM((SUP,), jnp.float32),
            pltpu.VMEM((SUP,), jnp.float32),
            pltpu.VMEM((SUP, F), jnp.float32),
            pltpu.VMEM((SUP, F), jnp.float32),
            pltpu.VMEM((zb, F), jnp.float32),
            pltpu.VMEM((96,), jnp.float32),
            pltpu.VMEM_SHARED((n_nodes, F), jnp.float32),
            pltpu.SemaphoreType.DMA,
            pltpu.SemaphoreType.DMA,
            pltpu.SemaphoreType.DMA,
            pltpu.SemaphoreType.DMA,
        ],
        compiler_params=pltpu.CompilerParams(use_tc_tiling_on_sc=False),
    )
    def edge_pass(p_hbm, e0_hbm, e1_hbm, e2_hbm, w_hbm, src_hbm, dst_hbm,
                  out_hbm, sv0, sv1, dv0, dv1, ev00, ev01, ev10, ev11,
                  ev20, ev21, gv0, gv1, z_v, w_v, acc_sh, ls0, ls1, gs0, gs1):
        c = lax.axis_index("c")
        s = lax.axis_index("s")
        wid = s * NC + c
        sv = (sv0, sv1)
        dv = (dv0, dv1)
        ev = ((ev00, ev10, ev20), (ev01, ev11, ev21))
        gv = (gv0, gv1)
        ls = (ls0, ls1)
        gs = (gs0, gs1)
        e_hbm = (e0_hbm, e1_hbm, e2_hbm)

        pltpu.sync_copy(w_hbm, w_v)
        wvec = [[w_v[pl.ds(k * F + h * 16, 16)] for h in range(2)]
                for k in range(3)]

        def zfill(j, _):
            z_v[j, pl.ds(0, 16)] = jnp.zeros((16,), jnp.float32)
            z_v[j, pl.ds(16, 16)] = jnp.zeros((16,), jnp.float32)
            return 0
        lax.fori_loop(0, zb, zfill, 0)

        def zcopy(i, _):
            blk = s + i * NS

            @pl.when(blk < n_zb)
            def _():
                pltpu.sync_copy(z_v, acc_sh.at[pl.ds(blk * zb, zb)])
            return 0
        lax.fori_loop(0, zmax, zcopy, 0)
        plsc.subcore_barrier()

        def lin_cps(sup, b):
            base = sup * SUP
            cps = [pltpu.make_async_copy(
                src_hbm.at[pl.ds(base, SUP)], sv[b], ls[b])]
            for k in range(3):
                cps.append(pltpu.make_async_copy(
                    e_hbm[k].at[pl.ds(base, SUP)], ev[b][k], ls[b]))
            for j in range(NSUB):
                cps.append(pltpu.make_async_copy(
                    dst_hbm.at[pl.ds(base + j * CH, CH)], dv[b].at[j], ls[b]))
            return cps

        def g_cps(b):
            return [
                pltpu.make_async_copy(
                    p_hbm.at[sv[b].at[pl.ds(j * CH, CH)]],
                    gv[b].at[pl.ds(j * CH, CH)], gs[b])
                for j in range(NSUB)
            ]

        def stage(i, a):
            b = 1 - a
            sup_a = wid + i * NW

            @pl.when(sup_a < n_sup)
            def _():
                for cp in lin_cps(sup_a, a):
                    cp.wait()
                for cp in g_cps(a):
                    cp.start()

            sup_b = wid + (i - 1) * NW

            @pl.when(jnp.logical_and(i >= 1, sup_b < n_sup))
            def _():
                for cp in g_cps(b):
                    cp.wait()

                def fuse_body(g, _):
                    base16 = g * 16
                    e0v = ev[b][0][pl.ds(base16, 16)]
                    e1v = ev[b][1][pl.ds(base16, 16)]
                    e2v = ev[b][2][pl.ds(base16, 16)]
                    for jj in range(16):
                        j = base16 + jj
                        for h in range(2):
                            q = (e0v[jj] * wvec[0][h] + e1v[jj] * wvec[1][h]
                                 + e2v[jj] * wvec[2][h])
                            sl = pl.ds(h * 16, 16)
                            gv[b][j, sl] = jnp.maximum(gv[b][j, sl] + q, 0.0)
                    return 0
                lax.fori_loop(0, SUP // 16, fuse_body, 0)
                for j in range(NSUB):
                    pltpu.sync_copy(
                        gv[b].at[pl.ds(j * CH, CH)],
                        acc_sh.at[dv[b].at[j]], add=True)

            sup_c = wid + (i + 1) * NW

            @pl.when(sup_c < n_sup)
            def _():
                for cp in lin_cps(sup_c, b):
                    cp.start()

        @pl.when(wid < n_sup)
        def _():
            for cp in lin_cps(wid, 0):
                cp.start()

        def loop_body(k, _):
            stage(2 * k, 0)
            stage(2 * k + 1, 1)
            return 0
        lax.fori_loop(0, kloop, loop_body, 0)
        plsc.subcore_barrier()

        if n_pad > n_nodes:
            @pl.when(s == 0)
            def _():
                pltpu.sync_copy(
                    z_v.at[pl.ds(0, n_pad - n_nodes)],
                    out_hbm.at[pl.ds(c * n_pad + n_nodes, n_pad - n_nodes)])

        def fcopy(i, _):
            blk = s + i * NS

            @pl.when(blk < n_zb)
            def _():
                row0 = blk * zb
                pltpu.sync_copy(acc_sh.at[pl.ds(row0, zb)], z_v)
                pltpu.sync_copy(z_v, out_hbm.at[pl.ds(c * n_pad + row0, zb)])
            return 0
        lax.fori_loop(0, zmax, fcopy, 0)

    return edge_pass


def kernel(node_features, edge_features, edge_idx, batch_idx,
           WM1, bM1, WU1, bU1, WM2, bM2, WU2, bU2, Wh, bh, Wo, bo):
    n = node_features.shape[0]
    n_pad = -(-n // 32) * 32
    m4 = n_pad // 4

    eye4 = jnp.eye(4, dtype=jnp.float32)

    def bd4(w):
        return jnp.kron(eye4, w)

    def tile4(b):
        return jnp.tile(b, 4)[None, :]

    xp = jnp.pad(node_features, ((0, n_pad - n), (0, F - 21)))
    x_packed = xp.reshape(m4, 4 * F)

    e0 = edge_features[:, 0]
    e1 = edge_features[:, 1]
    e2 = edge_features[:, 2]
    src = edge_idx[0]
    dst = edge_idx[1]
    w96_1 = WM1[21:24].reshape(96)
    w96_2 = WM2[32:35].reshape(96)

    bfull = jnp.pad(batch_idx, (0, n_pad - n), constant_values=64)
    oh = tuple(
        (bfull[k::4][:, None] == jnp.arange(64)[None, :]).astype(jnp.float32)
        for k in range(4))

    wm1x = jnp.pad(WM1[:21], ((0, 11), (0, 0)))
    wu1x = jnp.pad(WU1[:21], ((0, 11), (0, 0)))

    edge_pass = _make_edge_pass(n, n_pad, edge_features.shape[0])

    p1 = _mm_bias(x_packed, bd4(wm1x), tile4(bM1))
    agg1 = edge_pass(p1.reshape(n_pad, F), e0, e1, e2, w96_1, src, dst)
    y, p2 = _update_and_project(
        x_packed, agg1.reshape(2, m4, 4 * F), bd4(wu1x), bd4(WU1[21:53]),
        tile4(bU1), bd4(WM2[:32]), tile4(bM2))

    agg2 = edge_pass(p2.reshape(n_pad, F), e0, e1, e2, w96_2, src, dst)

    return _final(
        y, agg2.reshape(2, m4, 4 * F), oh, bd4(WU2[:32]), bd4(WU2[32:64]),
        tile4(bU2), Wh, bh[None, :], Wo, bo[None, :])
